# Initial kernel scaffold; baseline (speedup 1.0000x reference)
import jax
import jax.numpy as jnp
from jax.experimental import pallas as pl


def kernel(h, Wq, bq, Wk, bk, Wv, bv):
    raise NotImplementedError("write your pallas kernel here")


# speedup vs baseline: 384.6958x; 384.6958x over previous
import functools

import jax
import jax.numpy as jnp
import numpy as np
from jax.experimental import pallas as pl
from jax.experimental.pallas import tpu as pltpu

_B, _T, _D = 2, 2048, 1024
_K = 512
_NT = jax.lax.dot_general

_SIGN = int(np.int32(np.uint32(0x80000000)))


def _attn_kernel(h_ref, wq_ref, bq_ref, wk_ref, wv_ref, bv_ref, out_ref):
    h = h_ref[0]
    h_last = h[_T - 1:_T, :]
    f32 = jnp.float32
    q = _NT(h_last, wq_ref[...], (((1,), (1,)), ((), ())),
            preferred_element_type=f32) + bq_ref[...]
    u = _NT(q, wk_ref[...], (((1,), (0,)), ((), ())), preferred_element_type=f32)
    s = _NT(u, h, (((1,), (1,)), ((), ())),
            preferred_element_type=f32) * f32(1.0 / np.sqrt(_D))

    xi = jax.lax.bitcast_convert_type(s, jnp.int32)
    key = xi ^ ((xi >> 31) & jnp.int32(0x7FFFFFFF))
    kk = jnp.int32(_K)
    cnt_pos = jnp.sum((key >= 0).astype(jnp.int32))
    take_pos = cnt_pos >= kk
    prefix = jnp.where(take_pos, jnp.int32(0), jnp.int32(_SIGN))
    kk = jnp.where(take_pos, kk, kk - cnt_pos)
    for bit in range(30, -1, -1):
        m = jnp.int32(np.int32(np.uint32((0xFFFFFFFF << (bit + 1)) & 0xFFFFFFFF)))
        bitc = jnp.int32(1 << bit)
        cnt = jnp.sum(((key & (m | bitc)) == (prefix | bitc)).astype(jnp.int32))
        take = cnt >= kk
        prefix = jnp.where(take, prefix | bitc, prefix)
        kk = jnp.where(take, kk, kk - cnt)
    sel = key >= prefix

    mx = jnp.max(s)
    w = jnp.where(sel, jnp.exp(s - mx), f32(0.0))
    denom = jnp.sum(w)
    acc = _NT(w, h, (((1,), (0,)), ((), ())), preferred_element_type=f32)
    out = _NT(acc, wv_ref[...], (((1,), (1,)), ((), ())),
              preferred_element_type=f32) / denom + bv_ref[...]
    out_ref[0] = out


@jax.jit
def kernel(h, Wq, bq, Wk, bk, Wv, bv):
    del bk
    bq2 = bq.reshape(1, _D)
    bv2 = bv.reshape(1, _D)
    grid = (_B,)
    out = pl.pallas_call(
        _attn_kernel,
        grid=grid,
        in_specs=[
            pl.BlockSpec((1, _T, _D), lambda b: (b, 0, 0)),
            pl.BlockSpec((_D, _D), lambda b: (0, 0)),
            pl.BlockSpec((1, _D), lambda b: (0, 0)),
            pl.BlockSpec((_D, _D), lambda b: (0, 0)),
            pl.BlockSpec((_D, _D), lambda b: (0, 0)),
            pl.BlockSpec((1, _D), lambda b: (0, 0)),
        ],
        out_specs=pl.BlockSpec((1, 1, _D), lambda b: (b, 0, 0)),
        out_shape=jax.ShapeDtypeStruct((_B, 1, _D), jnp.float32),
        compiler_params=pltpu.CompilerParams(
            dimension_semantics=("arbitrary",),
        ),
    )(h, Wq, bq2, Wk, Wv, bv2)
    return out.reshape(_B, _D)
